# baseline (speedup 1.0000x reference)
import jax
import jax.numpy as jnp
from jax.experimental import pallas as pl
from jax.experimental.pallas import tpu as pltpu

_H = 32
_W = 32
_HW = _H * _W
_HALF = 384
_DIM = 2 * _HALF


def _pos_body(row_ref, col_ref, pos_ref):
    f = jax.lax.broadcasted_iota(jnp.int32, (_HW, _H), 0)
    k = jax.lax.broadcasted_iota(jnp.int32, (_HW, _H), 1)
    m_row = (f // _W == k).astype(jnp.float32)
    m_col = (f % _W == k).astype(jnp.float32)
    dn = (((1,), (0,)), ((), ()))
    pos_ref[:, :_HALF] = jax.lax.dot_general(
        m_row, row_ref[...], dn, preferred_element_type=jnp.float32)
    pos_ref[:, _HALF:] = jax.lax.dot_general(
        m_col, col_ref[...], dn, preferred_element_type=jnp.float32)


def _add_body(x_ref, pos_ref, o_ref):
    o_ref[...] = x_ref[...] + pos_ref[...][None]


def kernel(x, row_table, col_table):
    n, c, h, w = x.shape
    xt = jnp.transpose(x, (0, 2, 3, 1)).reshape(n, h * w, c)
    pos = pl.pallas_call(
        _pos_body,
        out_shape=jax.ShapeDtypeStruct((h * w, c), jnp.float32),
    )(row_table, col_table)
    nb = 2
    out = pl.pallas_call(
        _add_body,
        grid=(n // nb,),
        in_specs=[
            pl.BlockSpec((nb, h * w, c), lambda b: (b, 0, 0)),
            pl.BlockSpec((h * w, c), lambda b: (0, 0)),
        ],
        out_specs=pl.BlockSpec((nb, h * w, c), lambda b: (b, 0, 0)),
        out_shape=jax.ShapeDtypeStruct((n, h * w, c), x.dtype),
        compiler_params=pltpu.CompilerParams(
            dimension_semantics=("parallel",)),
    )(xt, pos)
    return jnp.transpose(out.reshape(n, h, w, c), (0, 3, 1, 2))
